# baseline (speedup 1.0000x reference)
import functools

import jax
import jax.numpy as jnp
from jax import lax
from jax.experimental import pallas as pl
from jax.experimental.pallas import tpu as pltpu
from jax.experimental.pallas import tpu_sc as plsc

_POOL = 7
_SR = 2
_C = 256
_NU = _C // 32
_N = 1000
_NW = 32
_R = 32
_RPY = _POOL * 16

_HSF = (128.0, 64.0, 32.0, 16.0)
_WSF = (128.0, 64.0, 32.0, 16.0)
_SCALES = (0.25, 0.125, 0.0625, 0.03125)
_OFFS = (0.0, 16384.0, 20480.0, 21504.0)


def _sc_body(table, params, out,
             pbuf, idxb0, idxb1, wb0, wb1, rows0, rows1, obuf0, obuf1,
             sem0, sem1, osem0, osem1):
    wid = lax.axis_index("s") * 2 + lax.axis_index("c")
    base = wid * _R
    pltpu.sync_copy(params.at[pl.ds(base * 16, _R * 16)], pbuf)
    nthis = jnp.minimum(_R, _N - base)

    lane = lax.iota(jnp.int32, 16)
    idxbs = (idxb0, idxb1)
    wbs = (wb0, wb1)
    rows = (rows0, rows1)
    sems = (sem0, sem1)
    obufs = (obuf0, obuf1)
    osems = (osem0, osem1)

    def roi_body(k, carry):
        pv = pbuf[pl.ds(k * 16, 16)]
        rx1 = pv[0]
        ry1 = pv[1]
        bw = pv[2]
        bh = pv[3]
        wf = pv[4]
        hf = pv[5]
        off = pv[6].astype(jnp.int32)
        wi = pv[4].astype(jnp.int32)

        def axis_lanes(sample_f, start, binsz, lim, corner_hi):
            ps = start + ((sample_f + 0.5) * 0.5) * binsz
            m = jnp.where((ps >= -1.0) & (ps <= lim), 1.0, 0.0)
            pc = jnp.maximum(ps, 0.0)
            pl0 = pc.astype(jnp.int32).astype(jnp.float32)
            cond = pl0 >= lim - 1.0
            lo = jnp.where(cond, lim - 1.0, pl0)
            hi = jnp.where(cond, lim - 1.0, pl0 + 1.0)
            frac = jnp.where(cond, lim - 1.0, pc) - lo
            pos = jnp.where(corner_hi, hi, lo).astype(jnp.int32)
            wgt = jnp.where(corner_hi, frac, 1.0 - frac) * m
            return pos, wgt

        def build(py, dst_i, dst_w):
            def bbody(px, _):
                iyf = (2 * py + ((lane >> 3) & 1)).astype(jnp.float32)
                ixf = (2 * px + ((lane >> 1) & 1)).astype(jnp.float32)
                cy_hi = ((lane >> 2) & 1) == 1
                cx_hi = (lane & 1) == 1
                posy, wy16 = axis_lanes(iyf, ry1, bh, hf, cy_hi)
                posx, wx16 = axis_lanes(ixf, rx1, bw, wf, cx_hi)
                dst_i[pl.ds(px * 16, 16)] = off + posy * wi + posx
                dst_w[pl.ds(px * 16, 16)] = wy16 * wx16 * 0.25
                return 0
            lax.fori_loop(0, _POOL, bbody, 0)

        roi = base + k
        build(0, idxbs[0], wbs[0])
        cp = pltpu.async_copy(table.at[idxbs[0]], rows[0], sems[0])
        mhi = jnp.int32(-65536)
        ocps = []
        for py in range(_POOL):
            bi = py % 2
            if py + 1 < _POOL:
                nbi = (py + 1) % 2
                build(py + 1, idxbs[nbi], wbs[nbi])
                ncp = pltpu.async_copy(table.at[idxbs[nbi]], rows[nbi], sems[nbi])
            cp.wait()
            rb = rows[bi]
            wbuf = wbs[bi]
            ob = obufs[py % 2]
            if py >= 2:
                ocps[py - 2].wait()
            else:
                @pl.when(k > 0)
                def _():
                    pltpu.make_async_copy(ob, out.at[roi, py],
                                          osems[py % 2]).wait()

            def pxbody(px, _):
                rbase = px * 16
                wv = wbuf[pl.ds(rbase, 16)]
                zero = jnp.zeros((16,), jnp.float32)
                acc = [zero] * (2 * _NU)
                for j in range(16):
                    w = wv[j]
                    r = rbase + j
                    for u in range(_NU):
                        word = rb[r, pl.ds(u * 16, 16)]
                        f0 = lax.bitcast_convert_type(word << 16, jnp.float32)
                        f1 = lax.bitcast_convert_type(word & mhi, jnp.float32)
                        acc[u] = acc[u] + w * f0
                        acc[_NU + u] = acc[_NU + u] + w * f1
                for v in range(2 * _NU):
                    ob[px, pl.ds(v * 16, 16)] = acc[v]
                return 0

            lax.fori_loop(0, _POOL, pxbody, 0)
            ocps.append(pltpu.async_copy(ob, out.at[roi, py], osems[py % 2]))
            if py + 1 < _POOL:
                cp = ncp
        return carry

    lax.fori_loop(0, nthis, roi_body, 0)
    pltpu.make_async_copy(obufs[1], out.at[base, 5], osems[1]).wait()
    pltpu.make_async_copy(obufs[0], out.at[base, 6], osems[0]).wait()


def _tr_body(x_ref, o_ref):
    x = x_ref[...]
    k = _POOL * _POOL
    eye = (lax.broadcasted_iota(jnp.int32, (k, k), 0) ==
           lax.broadcasted_iota(jnp.int32, (k, k), 1)).astype(jnp.float32)
    o_ref[...] = lax.dot_general(
        x, eye, dimension_numbers=(((1,), (0,)), ((), ())),
        preferred_element_type=jnp.float32)


_TRB = 8


def _get_tc_tr():
    global _tc_tr
    if _tc_tr is None:
        _tc_tr = pl.pallas_call(
            _tr_body,
            grid=(_N // _TRB,),
            in_specs=[pl.BlockSpec((_TRB, _POOL * _POOL, _C),
                                   lambda i: (i, 0, 0))],
            out_specs=pl.BlockSpec((_TRB, _C, _POOL * _POOL),
                                   lambda i: (i, 0, 0)),
            out_shape=jax.ShapeDtypeStruct((_N, _C, _POOL * _POOL),
                                           jnp.float32),
        )
    return _tc_tr


_tc_tr = None
_sc_call = None


def _get_sc_call():
    global _sc_call
    if _sc_call is None:
        mesh = plsc.VectorSubcoreMesh(core_axis_name="c", subcore_axis_name="s")
        _sc_call = pl.kernel(
            _sc_body,
            out_type=jax.ShapeDtypeStruct((_N, _POOL, _POOL, _C), jnp.float32),
            mesh=mesh,
            scratch_types=[
                pltpu.VMEM((_R * 16,), jnp.float32),
                pltpu.VMEM((_RPY,), jnp.int32),
                pltpu.VMEM((_RPY,), jnp.int32),
                pltpu.VMEM((_RPY,), jnp.float32),
                pltpu.VMEM((_RPY,), jnp.float32),
                pltpu.VMEM((_RPY, _C // 2), jnp.int32),
                pltpu.VMEM((_RPY, _C // 2), jnp.int32),
                pltpu.VMEM((_POOL, _C), jnp.float32),
                pltpu.VMEM((_POOL, _C), jnp.float32),
                pltpu.SemaphoreType.DMA,
                pltpu.SemaphoreType.DMA,
                pltpu.SemaphoreType.DMA,
                pltpu.SemaphoreType.DMA,
            ],
        )
    return _sc_call


def kernel(p2, p3, p4, p5, proposals, img_shapes):
    c = p2.shape[1]
    table = jnp.concatenate(
        [p2[0].reshape(c, -1), p3[0].reshape(c, -1),
         p4[0].reshape(c, -1), p5[0].reshape(c, -1)], axis=1).T
    tb = table.astype(jnp.bfloat16)
    packed = jax.lax.bitcast_convert_type(
        jnp.stack([tb[:, :c // 2], tb[:, c // 2:]], axis=-1), jnp.int32)

    x1, y1, x2, y2 = (proposals[:, 0], proposals[:, 1],
                      proposals[:, 2], proposals[:, 3])
    area = (x2 - x1) * (y2 - y1)
    lvl = jnp.floor(4.0 + jnp.log2(jnp.sqrt(area) / 224.0 + 1e-6))
    lvl = jnp.clip(lvl, 2.0, 5.0).astype(jnp.int32) - 2
    scale = jnp.asarray(_SCALES, jnp.float32)[lvl]
    wf = jnp.asarray(_WSF, jnp.float32)[lvl]
    hf = jnp.asarray(_HSF, jnp.float32)[lvl]
    off = jnp.asarray(_OFFS, jnp.float32)[lvl]
    rx1 = x1 * scale
    ry1 = y1 * scale
    bw = jnp.maximum(x2 * scale - rx1, 1.0) / _POOL
    bh = jnp.maximum(y2 * scale - ry1, 1.0) / _POOL
    zero = jnp.zeros_like(off)
    params = jnp.stack([rx1, ry1, bw, bh, wf, hf, off] + [zero] * 9, axis=1)
    params = jnp.concatenate(
        [params, jnp.zeros((_NW * _R - _N, 16), jnp.float32)],
        axis=0).reshape(-1)

    out = _get_sc_call()(packed, params)
    out = _get_tc_tr()(out.reshape(_N, _POOL * _POOL, _C))
    return out.reshape(_N, _C, _POOL, _POOL)


# speedup vs baseline: 1.1131x; 1.1131x over previous
import functools

import jax
import jax.numpy as jnp
from jax import lax
from jax.experimental import pallas as pl
from jax.experimental.pallas import tpu as pltpu
from jax.experimental.pallas import tpu_sc as plsc

_POOL = 7
_SR = 2
_C = 256
_NU = _C // 32
_N = 1000
_NW = 32
_R = 32
_RPY = _POOL * 16

_HSF = (128.0, 64.0, 32.0, 16.0)
_WSF = (128.0, 64.0, 32.0, 16.0)
_SCALES = (0.25, 0.125, 0.0625, 0.03125)
_OFFS = (0.0, 16384.0, 20480.0, 21504.0)


def _sc_body(table, params, out,
             pbuf, idxb0, idxb1, wb0, wb1, rows0, rows1, obuf0, obuf1,
             sem0, sem1, osem0, osem1):
    wid = lax.axis_index("s") * 2 + lax.axis_index("c")
    base = wid * _R
    pltpu.sync_copy(params.at[pl.ds(base * 16, _R * 16)], pbuf)
    nthis = jnp.minimum(_R, _N - base)

    lane = lax.iota(jnp.int32, 16)
    idxbs = (idxb0, idxb1)
    wbs = (wb0, wb1)
    rows = (rows0, rows1)
    sems = (sem0, sem1)
    obufs = (obuf0, obuf1)
    osems = (osem0, osem1)

    def roi_body(k, carry):
        pv = pbuf[pl.ds(k * 16, 16)]
        rx1 = pv[0]
        ry1 = pv[1]
        bw = pv[2]
        bh = pv[3]
        wf = pv[4]
        hf = pv[5]
        off = pv[6].astype(jnp.int32)
        wi = pv[4].astype(jnp.int32)

        def axis_lanes(sample_f, start, binsz, lim, corner_hi):
            ps = start + ((sample_f + 0.5) * 0.5) * binsz
            m = jnp.where((ps >= -1.0) & (ps <= lim), 1.0, 0.0)
            pc = jnp.maximum(ps, 0.0)
            pl0 = pc.astype(jnp.int32).astype(jnp.float32)
            cond = pl0 >= lim - 1.0
            lo = jnp.where(cond, lim - 1.0, pl0)
            hi = jnp.where(cond, lim - 1.0, pl0 + 1.0)
            frac = jnp.where(cond, lim - 1.0, pc) - lo
            pos = jnp.where(corner_hi, hi, lo).astype(jnp.int32)
            wgt = jnp.where(corner_hi, frac, 1.0 - frac) * m
            return pos, wgt

        def build(py, dst_i, dst_w):
            def bbody(px, _):
                iyf = (2 * py + ((lane >> 3) & 1)).astype(jnp.float32)
                ixf = (2 * px + ((lane >> 1) & 1)).astype(jnp.float32)
                cy_hi = ((lane >> 2) & 1) == 1
                cx_hi = (lane & 1) == 1
                posy, wy16 = axis_lanes(iyf, ry1, bh, hf, cy_hi)
                posx, wx16 = axis_lanes(ixf, rx1, bw, wf, cx_hi)
                dst_i[pl.ds(px * 16, 16)] = off + posy * wi + posx
                dst_w[pl.ds(px * 16, 16)] = wy16 * wx16 * 0.25
                return 0
            lax.fori_loop(0, _POOL, bbody, 0)

        roi = base + k
        build(0, idxbs[0], wbs[0])
        cp = pltpu.async_copy(table.at[idxbs[0]], rows[0], sems[0])
        mhi = jnp.int32(-65536)
        ocps = []
        for py in range(_POOL):
            bi = py % 2
            if py + 1 < _POOL:
                nbi = (py + 1) % 2
                build(py + 1, idxbs[nbi], wbs[nbi])
                ncp = pltpu.async_copy(table.at[idxbs[nbi]], rows[nbi], sems[nbi])
            cp.wait()
            rb = rows[bi]
            wbuf = wbs[bi]
            ob = obufs[py % 2]
            if py >= 2:
                ocps[py - 2].wait()
            else:
                @pl.when(k > 0)
                def _():
                    pltpu.make_async_copy(ob, out.at[roi, py],
                                          osems[py % 2]).wait()

            def pxbody(px, _):
                rbase = px * 16
                wv = wbuf[pl.ds(rbase, 16)]
                zero = jnp.zeros((16,), jnp.float32)
                acc = [zero] * (2 * _NU)
                for j in range(16):
                    w = wv[j]
                    r = rbase + j
                    for u in range(_NU):
                        word = rb[r, pl.ds(u * 16, 16)]
                        f0 = lax.bitcast_convert_type(word << 16, jnp.float32)
                        f1 = lax.bitcast_convert_type(word & mhi, jnp.float32)
                        acc[u] = acc[u] + w * f0
                        acc[_NU + u] = acc[_NU + u] + w * f1
                for v in range(2 * _NU):
                    ob[px, pl.ds(v * 16, 16)] = acc[v]
                return 0

            lax.fori_loop(0, _POOL, pxbody, 0)
            ocps.append(pltpu.async_copy(ob, out.at[roi, py], osems[py % 2]))
            if py + 1 < _POOL:
                cp = ncp
        return carry

    lax.fori_loop(0, nthis, roi_body, 0)
    pltpu.make_async_copy(obufs[1], out.at[base, 5], osems[1]).wait()
    pltpu.make_async_copy(obufs[0], out.at[base, 6], osems[0]).wait()


def _tr_body(x_ref, o_ref):
    x = x_ref[...]
    k = _POOL * _POOL
    eye = (lax.broadcasted_iota(jnp.int32, (k, k), 0) ==
           lax.broadcasted_iota(jnp.int32, (k, k), 1)).astype(jnp.float32)
    o_ref[...] = lax.dot_general(
        x, eye, dimension_numbers=(((1,), (0,)), ((), ())),
        preferred_element_type=jnp.float32)


_TRB = 8


def _get_tc_tr():
    global _tc_tr
    if _tc_tr is None:
        _tc_tr = pl.pallas_call(
            _tr_body,
            grid=(_N // _TRB,),
            in_specs=[pl.BlockSpec((_TRB, _POOL * _POOL, _C),
                                   lambda i: (i, 0, 0))],
            out_specs=pl.BlockSpec((_TRB, _C, _POOL * _POOL),
                                   lambda i: (i, 0, 0)),
            out_shape=jax.ShapeDtypeStruct((_N, _C, _POOL * _POOL),
                                           jnp.float32),
        )
    return _tc_tr


_tc_tr = None
_sc_call = None


def _get_sc_call():
    global _sc_call
    if _sc_call is None:
        mesh = plsc.VectorSubcoreMesh(core_axis_name="c", subcore_axis_name="s")
        _sc_call = pl.kernel(
            _sc_body,
            out_type=jax.ShapeDtypeStruct((_N, _POOL, _POOL, _C), jnp.float32),
            mesh=mesh,
            scratch_types=[
                pltpu.VMEM((_R * 16,), jnp.float32),
                pltpu.VMEM((_RPY,), jnp.int32),
                pltpu.VMEM((_RPY,), jnp.int32),
                pltpu.VMEM((_RPY,), jnp.float32),
                pltpu.VMEM((_RPY,), jnp.float32),
                pltpu.VMEM((_RPY, _C // 2), jnp.int32),
                pltpu.VMEM((_RPY, _C // 2), jnp.int32),
                pltpu.VMEM((_POOL, _C), jnp.float32),
                pltpu.VMEM((_POOL, _C), jnp.float32),
                pltpu.SemaphoreType.DMA,
                pltpu.SemaphoreType.DMA,
                pltpu.SemaphoreType.DMA,
                pltpu.SemaphoreType.DMA,
            ],
        )
    return _sc_call


def kernel(p2, p3, p4, p5, proposals, img_shapes):
    c = p2.shape[1]
    table = jnp.concatenate(
        [p2[0].reshape(c, -1), p3[0].reshape(c, -1),
         p4[0].reshape(c, -1), p5[0].reshape(c, -1)], axis=1).T
    tb = table.astype(jnp.bfloat16)
    packed = jax.lax.bitcast_convert_type(
        jnp.stack([tb[:, :c // 2], tb[:, c // 2:]], axis=-1), jnp.int32)

    x1, y1, x2, y2 = (proposals[:, 0], proposals[:, 1],
                      proposals[:, 2], proposals[:, 3])
    area = (x2 - x1) * (y2 - y1)
    lvl = jnp.floor(4.0 + jnp.log2(jnp.sqrt(area) / 224.0 + 1e-6))
    lvl = jnp.clip(lvl, 2.0, 5.0).astype(jnp.int32) - 2
    scale = jnp.asarray(_SCALES, jnp.float32)[lvl]
    wf = jnp.asarray(_WSF, jnp.float32)[lvl]
    hf = jnp.asarray(_HSF, jnp.float32)[lvl]
    off = jnp.asarray(_OFFS, jnp.float32)[lvl]
    rx1 = x1 * scale
    ry1 = y1 * scale
    bw = jnp.maximum(x2 * scale - rx1, 1.0) / _POOL
    bh = jnp.maximum(y2 * scale - ry1, 1.0) / _POOL
    zero = jnp.zeros_like(off)
    params = jnp.stack([rx1, ry1, bw, bh, wf, hf, off] + [zero] * 9, axis=1)
    params = jnp.concatenate(
        [params, jnp.zeros((_NW * _R - _N, 16), jnp.float32)],
        axis=0).reshape(-1)

    out = _get_sc_call()(packed, params)
    return jnp.transpose(out, (0, 3, 1, 2))


# speedup vs baseline: 1.6379x; 1.4714x over previous
import functools

import jax
import jax.numpy as jnp
from jax import lax
from jax.experimental import pallas as pl
from jax.experimental.pallas import tpu as pltpu
from jax.experimental.pallas import tpu_sc as plsc

_POOL = 7
_SR = 2
_C = 256
_NU = _C // 32
_N = 1000
_NW = 32
_R = 32
_RPY = _POOL * 16

_C0_PY, _C1_PY = 4, 3
_C0_B, _C1_B = 28, 21
_C0_E, _C1_E = _C0_B * _C, _C1_B * _C
_OUT_E = _POOL * _POOL * _C

_HSF = (128.0, 64.0, 32.0, 16.0)
_WSF = (128.0, 64.0, 32.0, 16.0)
_SCALES = (0.25, 0.125, 0.0625, 0.03125)
_OFFS = (0.0, 16384.0, 20480.0, 21504.0)


def _sc_body(table, params, out,
             pbuf, idx0, idx1, w0, w1, rows0, rows1, ob0, ob1,
             gsem0, gsem1, osem0, osem1):
    wid = lax.axis_index("s") * 2 + lax.axis_index("c")
    base = wid * _R
    pltpu.sync_copy(params.at[pl.ds(base * 16, _R * 16)], pbuf)
    nthis = jnp.minimum(_R, _N - base)
    lane = lax.iota(jnp.int32, 16)
    mhi = jnp.int32(-65536)

    def scalars(k):
        pv = pbuf[pl.ds(k * 16, 16)]
        return (pv[0], pv[1], pv[2], pv[3], pv[4], pv[5],
                pv[6].astype(jnp.int32), pv[4].astype(jnp.int32))

    def axis_lanes(sample_f, start, binsz, lim, corner_hi):
        ps = start + ((sample_f + 0.5) * 0.5) * binsz
        m = jnp.where((ps >= -1.0) & (ps <= lim), 1.0, 0.0)
        pc = jnp.maximum(ps, 0.0)
        pl0 = pc.astype(jnp.int32).astype(jnp.float32)
        cond = pl0 >= lim - 1.0
        lo = jnp.where(cond, lim - 1.0, pl0)
        hi = jnp.where(cond, lim - 1.0, pl0 + 1.0)
        frac = jnp.where(cond, lim - 1.0, pc) - lo
        pos = jnp.where(corner_hi, hi, lo).astype(jnp.int32)
        wgt = jnp.where(corner_hi, frac, 1.0 - frac) * m
        return pos, wgt

    def build(sc, py_base, nbins, dst_i, dst_w):
        rx1, ry1, bw, bh, wf, hf, off, wi = sc

        def bbody(b, _):
            q = b // 7
            px = b - q * 7
            py = py_base + q
            iyf = (2 * py + ((lane >> 3) & 1)).astype(jnp.float32)
            ixf = (2 * px + ((lane >> 1) & 1)).astype(jnp.float32)
            cy_hi = ((lane >> 2) & 1) == 1
            cx_hi = (lane & 1) == 1
            posy, wy16 = axis_lanes(iyf, ry1, bh, hf, cy_hi)
            posx, wx16 = axis_lanes(ixf, rx1, bw, wf, cx_hi)
            dst_i[q, pl.ds(px * 16, 16)] = off + posy * wi + posx
            dst_w[pl.ds(b * 16, 16)] = wy16 * wx16 * 0.25
            return 0

        lax.fori_loop(0, nbins, bbody, 0)

    def gather(dst_i, rows_ref, sem, nq):
        for q in range(nq):
            pltpu.async_copy(table.at[dst_i.at[q]],
                             rows_ref.at[pl.ds(q * _RPY, _RPY)], sem)

    def wait_gather(dst_i, rows_ref, sem, nq):
        for q in range(nq):
            pltpu.make_async_copy(table.at[dst_i.at[q]],
                                  rows_ref.at[pl.ds(q * _RPY, _RPY)],
                                  sem).wait()

    def process(nbins, rows_ref, wref, ob):
        zero = jnp.zeros((16,), jnp.float32)

        def binbody(b, _):
            rbase = b * 16
            wv = wref[pl.ds(rbase, 16)]
            acc = [zero] * (2 * _NU)
            for j in range(16):
                w = wv[j]
                r = rbase + j
                for u in range(_NU):
                    word = rows_ref[r, pl.ds(u * 16, 16)]
                    f0 = lax.bitcast_convert_type(word << 16, jnp.float32)
                    f1 = lax.bitcast_convert_type(word & mhi, jnp.float32)
                    acc[u] = acc[u] + w * f0
                    acc[_NU + u] = acc[_NU + u] + w * f1
            for v in range(2 * _NU):
                ob[pl.ds(b * _C + v * 16, 16)] = acc[v]
            return 0

        lax.fori_loop(0, nbins, binbody, 0)

    build(scalars(0), 0, _C0_B, idx0, w0)
    gather(idx0, rows0, gsem0, _C0_PY)

    def roi_body(k, carry):
        roi = base + k
        sck = scalars(k)
        build(sck, _C0_PY, _C1_B, idx1, w1)
        gather(idx1, rows1, gsem1, _C1_PY)
        wait_gather(idx0, rows0, gsem0, _C0_PY)

        @pl.when(k > 0)
        def _():
            pltpu.make_async_copy(ob0, out.at[roi, pl.ds(0, _C0_E)],
                                  osem0).wait()

        process(_C0_B, rows0, w0, ob0)
        pltpu.async_copy(ob0, out.at[roi, pl.ds(0, _C0_E)], osem0)

        @pl.when(k + 1 < nthis)
        def _():
            build(scalars(k + 1), 0, _C0_B, idx0, w0)
            gather(idx0, rows0, gsem0, _C0_PY)

        wait_gather(idx1, rows1, gsem1, _C1_PY)

        @pl.when(k > 0)
        def _():
            pltpu.make_async_copy(ob1, out.at[roi, pl.ds(_C0_E, _C1_E)],
                                  osem1).wait()

        process(_C1_B, rows1, w1, ob1)
        pltpu.async_copy(ob1, out.at[roi, pl.ds(_C0_E, _C1_E)], osem1)
        return carry

    lax.fori_loop(0, nthis, roi_body, 0)
    pltpu.make_async_copy(ob0, out.at[base, pl.ds(0, _C0_E)], osem0).wait()
    pltpu.make_async_copy(ob1, out.at[base, pl.ds(_C0_E, _C1_E)],
                          osem1).wait()


_sc_call = None


def _get_sc_call():
    global _sc_call
    if _sc_call is None:
        mesh = plsc.VectorSubcoreMesh(core_axis_name="c", subcore_axis_name="s")
        _sc_call = pl.kernel(
            _sc_body,
            out_type=jax.ShapeDtypeStruct((_N, _OUT_E), jnp.float32),
            mesh=mesh,
            scratch_types=[
                pltpu.VMEM((_R * 16,), jnp.float32),
                pltpu.VMEM((_C0_PY, _RPY), jnp.int32),
                pltpu.VMEM((_C1_PY, _RPY), jnp.int32),
                pltpu.VMEM((_C0_B * 16,), jnp.float32),
                pltpu.VMEM((_C1_B * 16,), jnp.float32),
                pltpu.VMEM((_C0_B * 16, _C // 2), jnp.int32),
                pltpu.VMEM((_C1_B * 16, _C // 2), jnp.int32),
                pltpu.VMEM((_C0_E,), jnp.float32),
                pltpu.VMEM((_C1_E,), jnp.float32),
                pltpu.SemaphoreType.DMA,
                pltpu.SemaphoreType.DMA,
                pltpu.SemaphoreType.DMA,
                pltpu.SemaphoreType.DMA,
            ],
        )
    return _sc_call


def kernel(p2, p3, p4, p5, proposals, img_shapes):
    c = p2.shape[1]
    table = jnp.concatenate(
        [p2[0].reshape(c, -1), p3[0].reshape(c, -1),
         p4[0].reshape(c, -1), p5[0].reshape(c, -1)], axis=1).T
    tb = table.astype(jnp.bfloat16)
    packed = jax.lax.bitcast_convert_type(
        jnp.stack([tb[:, :c // 2], tb[:, c // 2:]], axis=-1), jnp.int32)

    x1, y1, x2, y2 = (proposals[:, 0], proposals[:, 1],
                      proposals[:, 2], proposals[:, 3])
    area = (x2 - x1) * (y2 - y1)
    lvl = jnp.floor(4.0 + jnp.log2(jnp.sqrt(area) / 224.0 + 1e-6))
    lvl = jnp.clip(lvl, 2.0, 5.0).astype(jnp.int32) - 2
    scale = jnp.asarray(_SCALES, jnp.float32)[lvl]
    wf = jnp.asarray(_WSF, jnp.float32)[lvl]
    hf = jnp.asarray(_HSF, jnp.float32)[lvl]
    off = jnp.asarray(_OFFS, jnp.float32)[lvl]
    rx1 = x1 * scale
    ry1 = y1 * scale
    bw = jnp.maximum(x2 * scale - rx1, 1.0) / _POOL
    bh = jnp.maximum(y2 * scale - ry1, 1.0) / _POOL
    zero = jnp.zeros_like(off)
    params = jnp.stack([rx1, ry1, bw, bh, wf, hf, off] + [zero] * 9, axis=1)
    params = jnp.concatenate(
        [params, jnp.zeros((_NW * _R - _N, 16), jnp.float32)],
        axis=0).reshape(-1)

    out = _get_sc_call()(packed, params)
    return jnp.transpose(out.reshape(_N, _POOL, _POOL, _C), (0, 3, 1, 2))


# speedup vs baseline: 1.6641x; 1.0160x over previous
import functools

import jax
import jax.numpy as jnp
from jax import lax
from jax.experimental import pallas as pl
from jax.experimental.pallas import tpu as pltpu
from jax.experimental.pallas import tpu_sc as plsc

_POOL = 7
_SR = 2
_C = 256
_NU = _C // 32
_N = 1000
_NW = 32
_R = 32
_RPY = _POOL * 16

_C0_PY, _C1_PY = 4, 3
_C0_B, _C1_B = 28, 21
_C0_E, _C1_E = _C0_B * _C, _C1_B * _C
_OUT_E = _POOL * _POOL * _C

_HSF = (128.0, 64.0, 32.0, 16.0)
_WSF = (128.0, 64.0, 32.0, 16.0)
_SCALES = (0.25, 0.125, 0.0625, 0.03125)
_OFFS = (0.0, 16384.0, 20480.0, 21504.0)


def _sc_body(table, params, out,
             pbuf, idx0, idx1, w0, w1, rows0, rows1, ob0, ob1,
             gsem0, gsem1, osem0, osem1):
    wid = lax.axis_index("s") * 2 + lax.axis_index("c")
    base = wid * _R
    pltpu.sync_copy(params.at[pl.ds(base * 16, _R * 16)], pbuf)
    nthis = jnp.minimum(_R, _N - base)
    lane = lax.iota(jnp.int32, 16)
    mhi = jnp.int32(-65536)

    def scalars(k):
        pv = pbuf[pl.ds(k * 16, 16)]
        return (pv[0], pv[1], pv[2], pv[3], pv[4], pv[5],
                pv[6].astype(jnp.int32), pv[4].astype(jnp.int32))

    def axis_lanes(sample_f, start, binsz, lim, corner_hi):
        ps = start + ((sample_f + 0.5) * 0.5) * binsz
        m = jnp.where((ps >= -1.0) & (ps <= lim), 1.0, 0.0)
        pc = jnp.maximum(ps, 0.0)
        pl0 = pc.astype(jnp.int32).astype(jnp.float32)
        cond = pl0 >= lim - 1.0
        lo = jnp.where(cond, lim - 1.0, pl0)
        hi = jnp.where(cond, lim - 1.0, pl0 + 1.0)
        frac = jnp.where(cond, lim - 1.0, pc) - lo
        pos = jnp.where(corner_hi, hi, lo).astype(jnp.int32)
        wgt = jnp.where(corner_hi, frac, 1.0 - frac) * m
        return pos, wgt

    def build(sc, py_base, nbins, dst_i, dst_w):
        rx1, ry1, bw, bh, wf, hf, off, wi = sc

        def bbody(b, _):
            q = b // 7
            px = b - q * 7
            py = py_base + q
            iyf = (2 * py + ((lane >> 3) & 1)).astype(jnp.float32)
            ixf = (2 * px + ((lane >> 1) & 1)).astype(jnp.float32)
            cy_hi = ((lane >> 2) & 1) == 1
            cx_hi = (lane & 1) == 1
            posy, wy16 = axis_lanes(iyf, ry1, bh, hf, cy_hi)
            posx, wx16 = axis_lanes(ixf, rx1, bw, wf, cx_hi)
            dst_i[q, pl.ds(px * 16, 16)] = off + posy * wi + posx
            dst_w[pl.ds(b * 16, 16)] = wy16 * wx16 * 0.25
            return 0

        lax.fori_loop(0, nbins, bbody, 0)

    def gather(dst_i, rows_ref, sem, nq):
        for q in range(nq):
            pltpu.async_copy(table.at[dst_i.at[q]],
                             rows_ref.at[pl.ds(q * _RPY, _RPY)], sem)

    def process(rows_ref, wref, ob, dst_i, sem, nq):
        zero = jnp.zeros((16,), jnp.float32)

        def binbody(b, _):
            rbase = b * 16
            wv = wref[pl.ds(rbase, 16)]
            acc = [zero] * (2 * _NU)
            for j in range(16):
                w = wv[j]
                r = rbase + j
                for u in range(_NU):
                    word = rows_ref[r, pl.ds(u * 16, 16)]
                    f0 = lax.bitcast_convert_type(word << 16, jnp.float32)
                    f1 = lax.bitcast_convert_type(word & mhi, jnp.float32)
                    acc[u] = acc[u] + w * f0
                    acc[_NU + u] = acc[_NU + u] + w * f1
            for v in range(2 * _NU):
                ob[pl.ds(b * _C + v * 16, 16)] = acc[v]
            return 0

        for q in range(nq):
            pltpu.make_async_copy(table.at[dst_i.at[q]],
                                  rows_ref.at[pl.ds(q * _RPY, _RPY)],
                                  sem).wait()
            lax.fori_loop(q * _POOL, (q + 1) * _POOL, binbody, 0)

    build(scalars(0), 0, _C0_B, idx0, w0)
    gather(idx0, rows0, gsem0, _C0_PY)

    def roi_body(k, carry):
        roi = base + k
        sck = scalars(k)
        build(sck, _C0_PY, _C1_B, idx1, w1)
        gather(idx1, rows1, gsem1, _C1_PY)

        @pl.when(k > 0)
        def _():
            pltpu.make_async_copy(ob0, out.at[roi, pl.ds(0, _C0_E)],
                                  osem0).wait()

        process(rows0, w0, ob0, idx0, gsem0, _C0_PY)
        pltpu.async_copy(ob0, out.at[roi, pl.ds(0, _C0_E)], osem0)

        @pl.when(k + 1 < nthis)
        def _():
            build(scalars(k + 1), 0, _C0_B, idx0, w0)
            gather(idx0, rows0, gsem0, _C0_PY)

        @pl.when(k > 0)
        def _():
            pltpu.make_async_copy(ob1, out.at[roi, pl.ds(_C0_E, _C1_E)],
                                  osem1).wait()

        process(rows1, w1, ob1, idx1, gsem1, _C1_PY)
        pltpu.async_copy(ob1, out.at[roi, pl.ds(_C0_E, _C1_E)], osem1)
        return carry

    lax.fori_loop(0, nthis, roi_body, 0)
    pltpu.make_async_copy(ob0, out.at[base, pl.ds(0, _C0_E)], osem0).wait()
    pltpu.make_async_copy(ob1, out.at[base, pl.ds(_C0_E, _C1_E)],
                          osem1).wait()


_sc_call = None


def _get_sc_call():
    global _sc_call
    if _sc_call is None:
        mesh = plsc.VectorSubcoreMesh(core_axis_name="c", subcore_axis_name="s")
        _sc_call = pl.kernel(
            _sc_body,
            out_type=jax.ShapeDtypeStruct((_N, _OUT_E), jnp.float32),
            mesh=mesh,
            scratch_types=[
                pltpu.VMEM((_R * 16,), jnp.float32),
                pltpu.VMEM((_C0_PY, _RPY), jnp.int32),
                pltpu.VMEM((_C1_PY, _RPY), jnp.int32),
                pltpu.VMEM((_C0_B * 16,), jnp.float32),
                pltpu.VMEM((_C1_B * 16,), jnp.float32),
                pltpu.VMEM((_C0_B * 16, _C // 2), jnp.int32),
                pltpu.VMEM((_C1_B * 16, _C // 2), jnp.int32),
                pltpu.VMEM((_C0_E,), jnp.float32),
                pltpu.VMEM((_C1_E,), jnp.float32),
                pltpu.SemaphoreType.DMA,
                pltpu.SemaphoreType.DMA,
                pltpu.SemaphoreType.DMA,
                pltpu.SemaphoreType.DMA,
            ],
        )
    return _sc_call


def kernel(p2, p3, p4, p5, proposals, img_shapes):
    c = p2.shape[1]
    table = jnp.concatenate(
        [p2[0].reshape(c, -1), p3[0].reshape(c, -1),
         p4[0].reshape(c, -1), p5[0].reshape(c, -1)], axis=1).T
    ui = jax.lax.bitcast_convert_type(table, jnp.uint32)

    def rne(u):
        return (u + 0x7FFF + ((u >> 16) & 1)) >> 16

    packed = jax.lax.bitcast_convert_type(
        rne(ui[:, :c // 2]) | (rne(ui[:, c // 2:]) << 16), jnp.int32)

    x1, y1, x2, y2 = (proposals[:, 0], proposals[:, 1],
                      proposals[:, 2], proposals[:, 3])
    area = (x2 - x1) * (y2 - y1)
    lvl = jnp.floor(4.0 + jnp.log2(jnp.sqrt(area) / 224.0 + 1e-6))
    lvl = jnp.clip(lvl, 2.0, 5.0).astype(jnp.int32) - 2
    scale = jnp.asarray(_SCALES, jnp.float32)[lvl]
    wf = jnp.asarray(_WSF, jnp.float32)[lvl]
    hf = jnp.asarray(_HSF, jnp.float32)[lvl]
    off = jnp.asarray(_OFFS, jnp.float32)[lvl]
    rx1 = x1 * scale
    ry1 = y1 * scale
    bw = jnp.maximum(x2 * scale - rx1, 1.0) / _POOL
    bh = jnp.maximum(y2 * scale - ry1, 1.0) / _POOL
    zero = jnp.zeros_like(off)
    params = jnp.stack([rx1, ry1, bw, bh, wf, hf, off] + [zero] * 9, axis=1)
    params = jnp.concatenate(
        [params, jnp.zeros((_NW * _R - _N, 16), jnp.float32)],
        axis=0).reshape(-1)

    out = _get_sc_call()(packed, params)
    return jnp.transpose(out.reshape(_N, _POOL, _POOL, _C), (0, 3, 1, 2))


# speedup vs baseline: 1.7972x; 1.0800x over previous
import functools

import jax
import jax.numpy as jnp
from jax import lax
from jax.experimental import pallas as pl
from jax.experimental.pallas import tpu as pltpu
from jax.experimental.pallas import tpu_sc as plsc

_POOL = 7
_SR = 2
_C = 256
_NU = _C // 32
_N = 1000
_NW = 32
_R = 32
_RPY = _POOL * 16

_C0_PY, _C1_PY = 4, 3
_C0_B, _C1_B = 28, 21
_C0_E, _C1_E = _C0_B * _C, _C1_B * _C
_OUT_E = _POOL * _POOL * _C

_HSF = (128.0, 64.0, 32.0, 16.0)
_WSF = (128.0, 64.0, 32.0, 16.0)
_SCALES = (0.25, 0.125, 0.0625, 0.03125)
_OFFS = (0.0, 16384.0, 20480.0, 21504.0)


def _sc_body(table, params, out,
             pbuf, idx0, idx1, w0, w1, rows0, rows1, ob0, ob1,
             gsem0, gsem1, osem0, osem1):
    wid = lax.axis_index("s") * 2 + lax.axis_index("c")
    base = wid * _R
    pltpu.sync_copy(params.at[pl.ds(base * 16, _R * 16)], pbuf)
    nthis = jnp.minimum(_R, _N - base)
    lane = lax.iota(jnp.int32, 16)
    mhi = jnp.int32(-65536)

    def scalars(k):
        pv = pbuf[pl.ds(k * 16, 16)]
        return (pv[0], pv[1], pv[2], pv[3], pv[4], pv[5],
                pv[6].astype(jnp.int32), pv[4].astype(jnp.int32))

    def axis_lanes(sample_f, start, binsz, lim, corner_hi):
        ps = start + ((sample_f + 0.5) * 0.5) * binsz
        m = jnp.where((ps >= -1.0) & (ps <= lim), 1.0, 0.0)
        pc = jnp.maximum(ps, 0.0)
        pl0 = pc.astype(jnp.int32).astype(jnp.float32)
        cond = pl0 >= lim - 1.0
        lo = jnp.where(cond, lim - 1.0, pl0)
        hi = jnp.where(cond, lim - 1.0, pl0 + 1.0)
        frac = jnp.where(cond, lim - 1.0, pc) - lo
        pos = jnp.where(corner_hi, hi, lo).astype(jnp.int32)
        wgt = jnp.where(corner_hi, frac, 1.0 - frac) * m
        return pos, wgt

    def build(sc, py_base, nbins, dst_i, dst_w):
        rx1, ry1, bw, bh, wf, hf, off, wi = sc

        def bbody(b, _):
            q = b // 7
            px = b - q * 7
            py = py_base + q
            iyf = (2 * py + ((lane >> 3) & 1)).astype(jnp.float32)
            ixf = (2 * px + ((lane >> 1) & 1)).astype(jnp.float32)
            cy_hi = ((lane >> 2) & 1) == 1
            cx_hi = (lane & 1) == 1
            posy, wy16 = axis_lanes(iyf, ry1, bh, hf, cy_hi)
            posx, wx16 = axis_lanes(ixf, rx1, bw, wf, cx_hi)
            dst_i[q, pl.ds(px * 16, 16)] = off + posy * wi + posx
            dst_w[pl.ds(b * 16, 16)] = wy16 * wx16 * 0.25
            return 0

        lax.fori_loop(0, nbins, bbody, 0)

    def gather(dst_i, rows_ref, sem, nq):
        for q in range(nq):
            pltpu.async_copy(table.at[dst_i.at[q]],
                             rows_ref.at[pl.ds(q * _RPY, _RPY)], sem)

    def process(rows_ref, wref, ob, dst_i, sem, nq):
        zero = jnp.zeros((16,), jnp.float32)

        def binbody(b, _):
            rbase = b * 16
            wv = wref[pl.ds(rbase, 16)]
            acc = [zero] * (2 * _NU)
            for j in range(16):
                w = wv[j]
                r = rbase + j
                for u in range(_NU):
                    word = rows_ref[r, pl.ds(u * 16, 16)]
                    f0 = lax.bitcast_convert_type(word << 16, jnp.float32)
                    f1 = lax.bitcast_convert_type(word, jnp.float32)
                    acc[u] = acc[u] + w * f0
                    acc[_NU + u] = acc[_NU + u] + w * f1
            for v in range(2 * _NU):
                ob[pl.ds(b * _C + v * 16, 16)] = acc[v]
            return 0

        for q in range(nq):
            pltpu.make_async_copy(table.at[dst_i.at[q]],
                                  rows_ref.at[pl.ds(q * _RPY, _RPY)],
                                  sem).wait()
            lax.fori_loop(q * _POOL, (q + 1) * _POOL, binbody, 0)

    build(scalars(0), 0, _C0_B, idx0, w0)
    gather(idx0, rows0, gsem0, _C0_PY)

    def roi_body(k, carry):
        roi = base + k
        sck = scalars(k)
        build(sck, _C0_PY, _C1_B, idx1, w1)
        gather(idx1, rows1, gsem1, _C1_PY)

        @pl.when(k > 0)
        def _():
            pltpu.make_async_copy(ob0, out.at[roi, pl.ds(0, _C0_E)],
                                  osem0).wait()

        process(rows0, w0, ob0, idx0, gsem0, _C0_PY)
        pltpu.async_copy(ob0, out.at[roi, pl.ds(0, _C0_E)], osem0)

        @pl.when(k + 1 < nthis)
        def _():
            build(scalars(k + 1), 0, _C0_B, idx0, w0)
            gather(idx0, rows0, gsem0, _C0_PY)

        @pl.when(k > 0)
        def _():
            pltpu.make_async_copy(ob1, out.at[roi, pl.ds(_C0_E, _C1_E)],
                                  osem1).wait()

        process(rows1, w1, ob1, idx1, gsem1, _C1_PY)
        pltpu.async_copy(ob1, out.at[roi, pl.ds(_C0_E, _C1_E)], osem1)
        return carry

    lax.fori_loop(0, nthis, roi_body, 0)
    pltpu.make_async_copy(ob0, out.at[base, pl.ds(0, _C0_E)], osem0).wait()
    pltpu.make_async_copy(ob1, out.at[base, pl.ds(_C0_E, _C1_E)],
                          osem1).wait()


_sc_call = None


def _get_sc_call():
    global _sc_call
    if _sc_call is None:
        mesh = plsc.VectorSubcoreMesh(core_axis_name="c", subcore_axis_name="s")
        _sc_call = pl.kernel(
            _sc_body,
            out_type=jax.ShapeDtypeStruct((_N, _OUT_E), jnp.float32),
            mesh=mesh,
            scratch_types=[
                pltpu.VMEM((_R * 16,), jnp.float32),
                pltpu.VMEM((_C0_PY, _RPY), jnp.int32),
                pltpu.VMEM((_C1_PY, _RPY), jnp.int32),
                pltpu.VMEM((_C0_B * 16,), jnp.float32),
                pltpu.VMEM((_C1_B * 16,), jnp.float32),
                pltpu.VMEM((_C0_B * 16, _C // 2), jnp.int32),
                pltpu.VMEM((_C1_B * 16, _C // 2), jnp.int32),
                pltpu.VMEM((_C0_E,), jnp.float32),
                pltpu.VMEM((_C1_E,), jnp.float32),
                pltpu.SemaphoreType.DMA,
                pltpu.SemaphoreType.DMA,
                pltpu.SemaphoreType.DMA,
                pltpu.SemaphoreType.DMA,
            ],
        )
    return _sc_call


def kernel(p2, p3, p4, p5, proposals, img_shapes):
    c = p2.shape[1]
    table = jnp.concatenate(
        [p2[0].reshape(c, -1), p3[0].reshape(c, -1),
         p4[0].reshape(c, -1), p5[0].reshape(c, -1)], axis=1).T
    ui = jax.lax.bitcast_convert_type(table, jnp.uint32)

    def rne(u):
        return (u + 0x7FFF + ((u >> 16) & 1)) >> 16

    packed = jax.lax.bitcast_convert_type(
        rne(ui[:, :c // 2]) | (rne(ui[:, c // 2:]) << 16), jnp.int32)

    x1, y1, x2, y2 = (proposals[:, 0], proposals[:, 1],
                      proposals[:, 2], proposals[:, 3])
    area = (x2 - x1) * (y2 - y1)
    lvl = jnp.floor(4.0 + jnp.log2(jnp.sqrt(area) / 224.0 + 1e-6))
    lvl = jnp.clip(lvl, 2.0, 5.0).astype(jnp.int32) - 2
    scale = jnp.asarray(_SCALES, jnp.float32)[lvl]
    wf = jnp.asarray(_WSF, jnp.float32)[lvl]
    hf = jnp.asarray(_HSF, jnp.float32)[lvl]
    off = jnp.asarray(_OFFS, jnp.float32)[lvl]
    rx1 = x1 * scale
    ry1 = y1 * scale
    bw = jnp.maximum(x2 * scale - rx1, 1.0) / _POOL
    bh = jnp.maximum(y2 * scale - ry1, 1.0) / _POOL
    zero = jnp.zeros_like(off)
    params = jnp.stack([rx1, ry1, bw, bh, wf, hf, off] + [zero] * 9, axis=1)
    params = jnp.concatenate(
        [params, jnp.zeros((_NW * _R - _N, 16), jnp.float32)],
        axis=0).reshape(-1)

    out = _get_sc_call()(packed, params)
    return jnp.transpose(out.reshape(_N, _POOL, _POOL, _C), (0, 3, 1, 2))


# speedup vs baseline: 1.7977x; 1.0002x over previous
import jax
import jax.numpy as jnp
from jax import lax
from jax.experimental import pallas as pl
from jax.experimental.pallas import tpu as pltpu
from jax.experimental.pallas import tpu_sc as plsc

_POOL = 7
_SR = 2
_C = 256
_NU = _C // 32
_N = 1000
_NW = 32
_R = 32
_RPY = _POOL * 16

_C0_PY, _C1_PY = 4, 3
_C0_B, _C1_B = 28, 21
_C0_E, _C1_E = _C0_B * _C, _C1_B * _C
_OUT_E = _POOL * _POOL * _C

_HSF = (128.0, 64.0, 32.0, 16.0)
_WSF = (128.0, 64.0, 32.0, 16.0)
_SCALES = (0.25, 0.125, 0.0625, 0.03125)
_OFFS = (0.0, 16384.0, 20480.0, 21504.0)


def _sc_body(table, params, out,
             pbuf, idx0, idx1, w0, w1, rows0, rows1, ob0, ob1,
             gsem0, gsem1, osem0, osem1):
    wid = lax.axis_index("s") * 2 + lax.axis_index("c")
    base = wid * _R
    pltpu.sync_copy(params.at[pl.ds(base * 16, _R * 16)], pbuf)
    nthis = jnp.minimum(_R, _N - base)
    lane = lax.iota(jnp.int32, 16)

    def scalars(k):
        pv = pbuf[pl.ds(k * 16, 16)]
        return (pv[0], pv[1], pv[2], pv[3], pv[4], pv[5],
                pv[6].astype(jnp.int32), pv[4].astype(jnp.int32))

    def axis_lanes(sample_f, start, binsz, lim, corner_hi):
        ps = start + ((sample_f + 0.5) * 0.5) * binsz
        m = jnp.where((ps >= -1.0) & (ps <= lim), 1.0, 0.0)
        pc = jnp.maximum(ps, 0.0)
        pl0 = pc.astype(jnp.int32).astype(jnp.float32)
        cond = pl0 >= lim - 1.0
        lo = jnp.where(cond, lim - 1.0, pl0)
        hi = jnp.where(cond, lim - 1.0, pl0 + 1.0)
        frac = jnp.where(cond, lim - 1.0, pc) - lo
        pos = jnp.where(corner_hi, hi, lo).astype(jnp.int32)
        wgt = jnp.where(corner_hi, frac, 1.0 - frac) * m
        return pos, wgt

    def build(sc, py_base, nbins, dst_i, dst_w):
        rx1, ry1, bw, bh, wf, hf, off, wi = sc

        def bbody(b, _):
            q = b // 7
            px = b - q * 7
            py = py_base + q
            iyf = (2 * py + ((lane >> 3) & 1)).astype(jnp.float32)
            ixf = (2 * px + ((lane >> 1) & 1)).astype(jnp.float32)
            cy_hi = ((lane >> 2) & 1) == 1
            cx_hi = (lane & 1) == 1
            posy, wy16 = axis_lanes(iyf, ry1, bh, hf, cy_hi)
            posx, wx16 = axis_lanes(ixf, rx1, bw, wf, cx_hi)
            dst_i[q, pl.ds(px * 16, 16)] = off + posy * wi + posx
            dst_w[pl.ds(b * 16, 16)] = wy16 * wx16 * 0.25
            return 0

        lax.fori_loop(0, nbins, bbody, 0)

    def gather(dst_i, rows_ref, sem, nq):
        for q in range(nq):
            pltpu.async_copy(table.at[dst_i.at[q]],
                             rows_ref.at[pl.ds(q * _RPY, _RPY)], sem)

    def process(rows_ref, wref, ob, dst_i, sem, nq):
        zero = jnp.zeros((16,), jnp.float32)

        def binbody(b, _):
            rbase = b * 16
            wv = wref[pl.ds(rbase, 16)]
            acc = [zero] * (2 * _NU)
            for j in range(16):
                w = wv[j]
                r = rbase + j
                for u in range(_NU):
                    word = rows_ref[r, pl.ds(u * 16, 16)]
                    f0 = lax.bitcast_convert_type(word << 16, jnp.float32)
                    f1 = lax.bitcast_convert_type(word, jnp.float32)
                    acc[u] = acc[u] + w * f0
                    acc[_NU + u] = acc[_NU + u] + w * f1
            for v in range(2 * _NU):
                ob[pl.ds(b * _C + v * 16, 16)] = acc[v]
            return 0

        for q in range(nq):
            pltpu.make_async_copy(table.at[dst_i.at[q]],
                                  rows_ref.at[pl.ds(q * _RPY, _RPY)],
                                  sem).wait()
            lax.fori_loop(q * _POOL, (q + 1) * _POOL, binbody, 0)

    build(scalars(0), 0, _C0_B, idx0, w0)
    gather(idx0, rows0, gsem0, _C0_PY)

    def roi_body(k, carry):
        roi = base + k
        sck = scalars(k)
        build(sck, _C0_PY, _C1_B, idx1, w1)
        gather(idx1, rows1, gsem1, _C1_PY)

        @pl.when(k > 0)
        def _():
            pltpu.make_async_copy(ob0, out.at[roi, pl.ds(0, _C0_E)],
                                  osem0).wait()

        process(rows0, w0, ob0, idx0, gsem0, _C0_PY)
        pltpu.async_copy(ob0, out.at[roi, pl.ds(0, _C0_E)], osem0)

        @pl.when(k + 1 < nthis)
        def _():
            build(scalars(k + 1), 0, _C0_B, idx0, w0)
            gather(idx0, rows0, gsem0, _C0_PY)

        @pl.when(k > 0)
        def _():
            pltpu.make_async_copy(ob1, out.at[roi, pl.ds(_C0_E, _C1_E)],
                                  osem1).wait()

        process(rows1, w1, ob1, idx1, gsem1, _C1_PY)
        pltpu.async_copy(ob1, out.at[roi, pl.ds(_C0_E, _C1_E)], osem1)
        return carry

    lax.fori_loop(0, nthis, roi_body, 0)
    pltpu.make_async_copy(ob0, out.at[base, pl.ds(0, _C0_E)], osem0).wait()
    pltpu.make_async_copy(ob1, out.at[base, pl.ds(_C0_E, _C1_E)],
                          osem1).wait()


_sc_call = None


def _get_sc_call():
    global _sc_call
    if _sc_call is None:
        mesh = plsc.VectorSubcoreMesh(core_axis_name="c", subcore_axis_name="s")
        _sc_call = pl.kernel(
            _sc_body,
            out_type=jax.ShapeDtypeStruct((_N, _OUT_E), jnp.float32),
            mesh=mesh,
            scratch_types=[
                pltpu.VMEM((_R * 16,), jnp.float32),
                pltpu.VMEM((_C0_PY, _RPY), jnp.int32),
                pltpu.VMEM((_C1_PY, _RPY), jnp.int32),
                pltpu.VMEM((_C0_B * 16,), jnp.float32),
                pltpu.VMEM((_C1_B * 16,), jnp.float32),
                pltpu.VMEM((_C0_B * 16, _C // 2), jnp.int32),
                pltpu.VMEM((_C1_B * 16, _C // 2), jnp.int32),
                pltpu.VMEM((_C0_E,), jnp.float32),
                pltpu.VMEM((_C1_E,), jnp.float32),
                pltpu.SemaphoreType.DMA,
                pltpu.SemaphoreType.DMA,
                pltpu.SemaphoreType.DMA,
                pltpu.SemaphoreType.DMA,
            ],
        )
    return _sc_call


def kernel(p2, p3, p4, p5, proposals, img_shapes):
    c = p2.shape[1]
    table = jnp.concatenate(
        [p2[0].reshape(c, -1), p3[0].reshape(c, -1),
         p4[0].reshape(c, -1), p5[0].reshape(c, -1)], axis=1).T
    ui = jax.lax.bitcast_convert_type(table, jnp.uint32)

    def rne(u):
        return (u + 0x7FFF + ((u >> 16) & 1)) >> 16

    packed = jax.lax.bitcast_convert_type(
        rne(ui[:, :c // 2]) | (rne(ui[:, c // 2:]) << 16), jnp.int32)

    x1, y1, x2, y2 = (proposals[:, 0], proposals[:, 1],
                      proposals[:, 2], proposals[:, 3])
    area = (x2 - x1) * (y2 - y1)
    lvl = jnp.floor(4.0 + jnp.log2(jnp.sqrt(area) / 224.0 + 1e-6))
    lvl = jnp.clip(lvl, 2.0, 5.0).astype(jnp.int32) - 2
    scale = jnp.asarray(_SCALES, jnp.float32)[lvl]
    wf = jnp.asarray(_WSF, jnp.float32)[lvl]
    hf = jnp.asarray(_HSF, jnp.float32)[lvl]
    off = jnp.asarray(_OFFS, jnp.float32)[lvl]
    rx1 = x1 * scale
    ry1 = y1 * scale
    bw = jnp.maximum(x2 * scale - rx1, 1.0) / _POOL
    bh = jnp.maximum(y2 * scale - ry1, 1.0) / _POOL
    zero = jnp.zeros_like(off)
    params = jnp.stack([rx1, ry1, bw, bh, wf, hf, off] + [zero] * 9, axis=1)
    params = jnp.concatenate(
        [params, jnp.zeros((_NW * _R - _N, 16), jnp.float32)],
        axis=0).reshape(-1)

    out = _get_sc_call()(packed, params)
    return jnp.transpose(out.reshape(_N, _POOL, _POOL, _C), (0, 3, 1, 2))
